# baseline (speedup 1.0000x reference)
import functools

import jax
import jax.numpy as jnp
from jax import lax
from jax.experimental import pallas as pl
from jax.experimental.pallas import tpu as pltpu
from jax.experimental.pallas import tpu_sc as plsc

_NUM_PATCHES = 2048
_NC = 2
_NS = 16
_CH = 64


def _cast_body(x_ref, o_ref):
    o_ref[...] = x_ref[...].astype(jnp.bfloat16)


def _cast(bytes):
    B, S, D = bytes.shape
    ST = 1024
    ns = S // ST
    return pl.pallas_call(
        _cast_body,
        grid=(B * ns,),
        in_specs=[pl.BlockSpec((1, ST, D), lambda i: (i // ns, i % ns, 0))],
        out_specs=pl.BlockSpec((1, ST, D), lambda i: (i // ns, i % ns, 0)),
        out_shape=jax.ShapeDtypeStruct((B, S, D), jnp.bfloat16),
    )(bytes)


_CH = 32
_PPT = 256


def _sc_body(bytes_hbm, idsf_hbm, zrow_hbm, sums_hbm,
             acc_v, rows_v, idbuf_v, idx_v, *, B, S, D, NP):
    c = lax.axis_index("c")
    s = lax.axis_index("s")
    w = c * _NS + s
    tiles_per_row = NP // _PPT
    row = w // tiles_per_row
    p0 = (w % tiles_per_row) * _PPT
    rbase = pl.multiple_of(row * S, 1024)
    DH = D // 2

    def cnt_chunk(cb, carry):
        alo, ahi = carry
        pltpu.sync_copy(idsf_hbm.at[pl.ds(rbase + cb * 1024, 1024)], idbuf_v)

        def cnt_vec(j, carry2):
            alo2, ahi2 = carry2
            v = idbuf_v[pl.ds(j * 16, 16)]
            one = jnp.ones((16,), jnp.int32)
            zero = jnp.zeros((16,), jnp.int32)
            alo2 = alo2 + jnp.where(v < p0, one, zero)
            ahi2 = ahi2 + jnp.where(v < p0 + _PPT, one, zero)
            return alo2, ahi2

        return lax.fori_loop(0, 1024 // 16, cnt_vec, (alo, ahi))

    z16 = jnp.zeros((16,), jnp.int32)
    alo, ahi = lax.fori_loop(0, S // 1024, cnt_chunk, (z16, z16))
    lo = alo[0]
    hi = ahi[0]
    for l in range(1, 16):
        lo = lo + alo[l]
        hi = hi + ahi[l]

    lo16 = (lo // 16) * 16
    ntp = hi - lo16
    nch = (ntp + _CH - 1) // _CH

    for h in range(2):
        pltpu.sync_copy(zrow_hbm, acc_v)

        def chunk(k, _):
            t0 = lo16 + k * _CH
            t0c = pl.multiple_of(jnp.minimum(t0, S - _CH), 16)
            off = t0 - t0c
            pltpu.sync_copy(idsf_hbm.at[pl.ds(rbase + t0c, _CH)], idx_v)
            pltpu.sync_copy(bytes_hbm.at[row, pl.ds(t0c, _CH), pl.ds(h * DH, DH)],
                            rows_v)

            for g in range(_CH // 16):
                idv = idx_v[pl.ds(g * 16, 16)]
                for l in range(16):
                    pos = k * _CH + g * 16 + l
                    rel0 = idv[l] - p0
                    ok = (pos < ntp) & (rel0 >= 0) & (rel0 < _PPT)
                    rel = jnp.where(ok, rel0, _PPT)
                    t = off + g * 16 + l
                    for j in range(DH // 16):
                        sl = pl.ds(j * 16, 16)
                        acc_v[rel, sl] = acc_v[rel, sl] + rows_v[t, sl]
            return 0

        lax.fori_loop(0, nch, chunk, 0)

        pltpu.sync_copy(acc_v.at[pl.ds(0, _PPT)],
                        sums_hbm.at[row, pl.ds(p0, _PPT), pl.ds(h * DH, DH)])


def _sc_segsum(bytes, patch_ids):
    B, S, D = bytes.shape
    NP = _NUM_PATCHES
    zrow = jnp.zeros((_PPT + 1, D // 2), jnp.float32)
    body = functools.partial(_sc_body, B=B, S=S, D=D, NP=NP)
    f = pl.kernel(
        body,
        out_type=jax.ShapeDtypeStruct((B, NP, D), jnp.float32),
        mesh=plsc.VectorSubcoreMesh(core_axis_name="c", subcore_axis_name="s"),
        scratch_types=[
            pltpu.VMEM((_PPT + 1, D // 2), jnp.float32),
            pltpu.VMEM((_CH, D // 2), jnp.float32),
            pltpu.VMEM((1024,), jnp.int32),
            pltpu.VMEM((_CH,), jnp.int32),
        ],
    )
    return f(bytes, patch_ids.reshape(-1).astype(jnp.int32), zrow)


def _proj_body(sums_ref, ids_ref, w_ref, b_ref, o_ref, *, PB, S):
    p = pl.program_id(1)
    p0 = p * PB

    def cnt_chunk(cb, cnt):
        ids = ids_ref[0, 0, pl.ds(cb * 1024, 1024)]
        patches = jax.lax.broadcasted_iota(jnp.int32, (PB, 1024), 0) + p0
        oh = (patches == ids[None, :]).astype(jnp.float32)
        return cnt + jnp.sum(oh, axis=1)

    cnt = lax.fori_loop(0, S // 1024, cnt_chunk, jnp.zeros((PB,), jnp.float32))
    cnt = jnp.maximum(cnt, 1.0)[:, None]
    mean = (sums_ref[0].astype(jnp.float32) / cnt).astype(jnp.bfloat16).astype(jnp.float32)
    o_ref[0] = lax.dot_general(
        mean, w_ref[...], (((1,), (0,)), ((), ())),
        preferred_element_type=jnp.float32) + b_ref[0][None, :]


def _proj(sums, ids3, W, b):
    B, NP, D = sums.shape
    S = ids3.shape[2]
    GD = W.shape[1]
    PB = 512
    npb = NP // PB
    body = functools.partial(_proj_body, PB=PB, S=S)
    return pl.pallas_call(
        body,
        grid=(B, npb),
        in_specs=[
            pl.BlockSpec((1, PB, D), lambda bb, pp: (bb, pp, 0)),
            pl.BlockSpec((1, 1, S), lambda bb, pp: (bb, 0, 0)),
            pl.BlockSpec((D, GD), lambda bb, pp: (0, 0)),
            pl.BlockSpec((1, GD), lambda bb, pp: (0, 0)),
        ],
        out_specs=pl.BlockSpec((1, PB, GD), lambda bb, pp: (bb, pp, 0)),
        out_shape=jax.ShapeDtypeStruct((B, NP, GD), jnp.float32),
    )(sums, ids3, W, b.reshape(1, GD))


def kernel(bytes, patch_ids, W, b):
    B, S, D = bytes.shape
    be = _cast(bytes)
    sums = _sc_segsum(bytes, patch_ids)
    pe = _proj(sums, patch_ids.reshape(B, 1, S).astype(jnp.int32), W, b)
    return (be, pe)
